# Initial kernel scaffold; baseline (speedup 1.0000x reference)
import jax
import jax.numpy as jnp
from jax.experimental import pallas as pl


def kernel(features, indices):
    raise NotImplementedError("write your pallas kernel here")


# speedup vs baseline: 2.4267x; 2.4267x over previous
import functools

import jax
import jax.numpy as jnp
from jax import lax
from jax.experimental import pallas as pl
from jax.experimental.pallas import tpu as pltpu
from jax.experimental.pallas import tpu_sc as plsc

_V = 262144
_GRID = (64, 64, 64)
_C = 16
_N = 2000000
_L = 16
_NR = 64
_VPR = _V // _NR
_BLK = 2048


def _read_scalar(vref, pos):
    return vref[pl.ds(pos, _L)][0]


def _sc_body(feats, idx_hbm, starts_hbm, out, starts_v, idx_v, feat_v, acc, cnt):
    w = lax.axis_index("s") * 2 + lax.axis_index("c")
    pltpu.sync_copy(starts_hbm, starts_v)
    zeros = jnp.zeros((_L,), jnp.float32)
    ones = jnp.ones((_L,), jnp.float32)
    lane = lax.iota(jnp.int32, _L)

    for rr in range(2):
        r = w * 2 + rr
        vbase = r * _VPR
        p0 = _read_scalar(starts_v, r)
        p1 = _read_scalar(starts_v, r + 1)

        def _zero(i, carry):
            cnt[pl.ds(i * _L, _L)] = zeros
            for c in range(_C):
                acc[pl.ds(c * _VPR + i * _L, _L)] = zeros
            return carry

        lax.fori_loop(0, _VPR // _L, _zero, 0)

        pa = (p0 // 8) * 8
        nblk = (p1 - pa + _BLK - 1) // _BLK

        def _block(b, carry):
            off0 = pa + b * _BLK
            off = pl.multiple_of(jnp.minimum(off0, _N - _BLK), 8)
            pltpu.sync_copy(idx_hbm.at[pl.ds(off, _BLK)], idx_v)
            for c in range(_C):
                pltpu.sync_copy(feats.at[pl.ds(c * _N + off, _BLK)],
                                feat_v.at[pl.ds(c * _BLK, _BLK)])
            lo = jnp.maximum(p0, off0)
            hi = jnp.minimum(p1, off0 + _BLK)

            def _group(j, carry2):
                s = pl.ds(j * _L, _L)
                g = off + j * _L + lane
                lidx = idx_v[s] - vbase
                m = ((g >= lo) & (g < hi)
                     & (lidx >= 0) & (lidx < _VPR))
                plsc.addupdate_scatter(cnt, [lidx], ones, mask=m)
                for c in range(_C):
                    fv = feat_v[pl.ds(c * _BLK + j * _L, _L)]
                    plsc.addupdate_scatter(acc, [lidx + (c * _VPR)], fv,
                                           mask=m)
                return carry2

            lax.fori_loop(0, _BLK // _L, _group, 0)
            return carry

        lax.fori_loop(0, nblk, _block, 0)

        def _mean(i, carry):
            rcp = 1.0 / jnp.maximum(cnt[pl.ds(i * _L, _L)], 1.0)
            for c in range(_C):
                s = pl.ds(c * _VPR + i * _L, _L)
                acc[s] = acc[s] * rcp
            return carry

        lax.fori_loop(0, _VPR // _L, _mean, 0)
        for c in range(_C):
            pltpu.sync_copy(acc.at[pl.ds(c * _VPR, _VPR)],
                            out.at[pl.ds(c * _V + vbase, _VPR)])


_mesh = plsc.VectorSubcoreMesh(core_axis_name="c", subcore_axis_name="s")

_voxelize = functools.partial(
    pl.kernel,
    mesh=_mesh,
    out_type=jax.ShapeDtypeStruct((_C * _V,), jnp.float32),
    compiler_params=pltpu.CompilerParams(needs_layout_passes=False),
    scratch_types=[
        pltpu.VMEM((96,), jnp.int32),
        pltpu.VMEM((_BLK,), jnp.int32),
        pltpu.VMEM((_C * _BLK,), jnp.float32),
        pltpu.VMEM((_C * _VPR,), jnp.float32),
        pltpu.VMEM((_VPR,), jnp.float32),
    ],
)(_sc_body)


@jax.jit
def kernel(features, indices):
    feats2d = features.reshape(_C * _N)
    idx = indices.astype(jnp.int32)
    bounds = jnp.arange(_NR, dtype=jnp.int32) * _VPR
    starts = jnp.searchsorted(idx, bounds, side="left").astype(jnp.int32)
    starts = jnp.concatenate([starts, jnp.full((32,), _N, jnp.int32)])
    out = _voxelize(feats2d, idx, starts)
    return out.reshape((1, _C) + _GRID)


# speedup vs baseline: 2.6826x; 1.1054x over previous
import functools

import jax
import jax.numpy as jnp
from jax import lax
from jax.experimental import pallas as pl
from jax.experimental.pallas import tpu as pltpu
from jax.experimental.pallas import tpu_sc as plsc

_V = 262144
_GRID = (64, 64, 64)
_C = 16
_N = 2000000
_L = 16
_NR = 64
_VPR = _V // _NR
_BLK = 1024
_GRP = _BLK // _L


def _read_scalar(vref, pos):
    return vref[pl.ds(pos, _L)][0]


def _sc_body(feats, idx_hbm, starts_hbm, out, starts_v, idx_v, feat_v, acc,
             cnt, sem):
    w = lax.axis_index("s") * 2 + lax.axis_index("c")
    pltpu.sync_copy(starts_hbm, starts_v)
    zeros = jnp.zeros((_L,), jnp.float32)
    ones = jnp.ones((_L,), jnp.float32)
    lane = lax.iota(jnp.int32, _L)

    def _issue(pa, b, buf):
        off = pl.multiple_of(jnp.minimum(pa + b * _BLK, _N - _BLK), 8)
        pltpu.async_copy(idx_hbm.at[pl.ds(off, _BLK)],
                         idx_v.at[pl.ds(buf * _BLK, _BLK)], sem)
        for c in range(_C):
            pltpu.async_copy(feats.at[pl.ds(c * _N + off, _BLK)],
                             feat_v.at[pl.ds((buf * _C + c) * _BLK, _BLK)],
                             sem)

    def _drain(buf):
        pltpu.make_async_copy(idx_hbm.at[pl.ds(0, _BLK)],
                              idx_v.at[pl.ds(buf * _BLK, _BLK)], sem).wait()
        pltpu.make_async_copy(feats.at[pl.ds(0, _C * _BLK)],
                              feat_v.at[pl.ds(buf * _C * _BLK, _C * _BLK)],
                              sem).wait()

    for rr in range(2):
        r = w * 2 + rr
        vbase = r * _VPR
        p0 = _read_scalar(starts_v, r)
        p1 = _read_scalar(starts_v, r + 1)

        def _zero(i, carry):
            cnt[pl.ds(i * _L, _L)] = zeros
            for c in range(_C):
                acc[pl.ds(c * _VPR + i * _L, _L)] = zeros
            return carry

        lax.fori_loop(0, _VPR // _L, _zero, 0)

        pa = (p0 // 8) * 8
        nblk = (p1 - pa + _BLK - 1) // _BLK
        npair = jnp.maximum((nblk + 1) // 2, 1)

        def _process(b, buf):
            off = pl.multiple_of(jnp.minimum(pa + b * _BLK, _N - _BLK), 8)
            lo = jnp.maximum(p0, pa + b * _BLK)
            hi = jnp.minimum(p1, pa + (b + 1) * _BLK)

            def _one_group(j):
                g = off + j * _L + lane
                lidx = idx_v[pl.ds(buf * _BLK + j * _L, _L)] - vbase
                m = ((g >= lo) & (g < hi)
                     & (lidx >= 0) & (lidx < _VPR))
                plsc.addupdate_scatter(cnt, [lidx], ones, mask=m)
                for c in range(_C):
                    fv = feat_v[pl.ds((buf * _C + c) * _BLK + j * _L, _L)]
                    plsc.addupdate_scatter(acc, [lidx + (c * _VPR)], fv,
                                           mask=m)

            def _group(j, carry2):
                _one_group(2 * j)
                _one_group(2 * j + 1)
                return carry2

            lax.fori_loop(0, _GRP // 2, _group, 0)

        def _pair(i, carry):
            b = 2 * i
            _issue(pa, b + 1, 1)
            _drain(0)
            _process(b, 0)
            _issue(pa, b + 2, 0)
            _drain(1)
            _process(b + 1, 1)
            return carry

        _issue(pa, 0, 0)
        lax.fori_loop(0, npair, _pair, 0)
        _drain(0)

        def _mean(i, carry):
            rcp = 1.0 / jnp.maximum(cnt[pl.ds(i * _L, _L)], 1.0)
            for c in range(_C):
                s = pl.ds(c * _VPR + i * _L, _L)
                acc[s] = acc[s] * rcp
            return carry

        lax.fori_loop(0, _VPR // _L, _mean, 0)
        for c in range(_C):
            pltpu.sync_copy(acc.at[pl.ds(c * _VPR, _VPR)],
                            out.at[pl.ds(c * _V + vbase, _VPR)])


_mesh = plsc.VectorSubcoreMesh(core_axis_name="c", subcore_axis_name="s")

_voxelize = functools.partial(
    pl.kernel,
    mesh=_mesh,
    out_type=jax.ShapeDtypeStruct((_C * _V,), jnp.float32),
    compiler_params=pltpu.CompilerParams(needs_layout_passes=False),
    scratch_types=[
        pltpu.VMEM((96,), jnp.int32),
        pltpu.VMEM((2 * _BLK,), jnp.int32),
        pltpu.VMEM((2 * _C * _BLK,), jnp.float32),
        pltpu.VMEM((_C * _VPR,), jnp.float32),
        pltpu.VMEM((_VPR,), jnp.float32),
        pltpu.SemaphoreType.DMA,
    ],
)(_sc_body)


@jax.jit
def kernel(features, indices):
    feats2d = features.reshape(_C * _N)
    idx = indices.astype(jnp.int32)
    bounds = jnp.arange(_NR, dtype=jnp.int32) * _VPR
    starts = jnp.searchsorted(idx, bounds, side="left").astype(jnp.int32)
    starts = jnp.concatenate([starts, jnp.full((32,), _N, jnp.int32)])
    out = _voxelize(feats2d, idx, starts)
    return out.reshape((1, _C) + _GRID)
